# baseline (device time: 19916 ns/iter reference)
import jax
import jax.numpy as jnp
from jax import lax
from jax.experimental import pallas as pl
from jax.experimental.pallas import tpu as pltpu

BM = 512


def _body(y_ref, x_ref, dy_ref, out_ref, acc_ref, comm_ref, send_sems, recv_sems):
    i = pl.program_id(0)
    n_steps = pl.num_programs(0)

    x = x_ref[...]
    dy = dy_ref[...]
    bm, d = x.shape
    ones_col = jnp.ones((d, 1), jnp.float32)
    ones_row = jnp.ones((2, bm), jnp.float32)
    s1 = jnp.dot(x, ones_col, preferred_element_type=jnp.float32)
    s2 = jnp.dot(x * x, ones_col, preferred_element_type=jnp.float32)
    mu = s1 * (1.0 / d)
    var = s2 * (1.0 / d) - mu * mu
    rstd = lax.rsqrt(var + 1e-5)
    g = dy * (rstd * x - rstd * mu)
    part = jnp.concatenate(
        [
            jnp.dot(ones_row[:1], g, preferred_element_type=jnp.float32),
            jnp.dot(ones_row[:1], dy, preferred_element_type=jnp.float32),
        ],
        axis=0,
    )

    @pl.when(i == 0)
    def _():
        acc_ref[...] = part

    @pl.when(i != 0)
    def _():
        acc_ref[...] += part

    @pl.when(i == n_steps - 1)
    def _():
        my_x = lax.axis_index("x")
        my_y = lax.axis_index("y")
        peers = (
            (1 - my_x, my_y),
            (my_x, 1 - my_y),
            (1 - my_x, 1 - my_y),
        )

        barrier = pltpu.get_barrier_semaphore()
        for peer in peers:
            pl.semaphore_signal(
                barrier, inc=1, device_id=peer,
                device_id_type=pl.DeviceIdType.MESH,
            )
        pl.semaphore_wait(barrier, 3)

        rdmas = []
        for k, peer in enumerate(peers):
            rdma = pltpu.make_async_remote_copy(
                src_ref=acc_ref,
                dst_ref=comm_ref.at[k],
                send_sem=send_sems.at[k],
                recv_sem=recv_sems.at[k],
                device_id=peer,
                device_id_type=pl.DeviceIdType.MESH,
            )
            rdma.start()
            rdmas.append(rdma)
        for rdma in rdmas:
            rdma.wait()

        out_ref[...] = (
            (acc_ref[...] + comm_ref[0]) + (comm_ref[1] + comm_ref[2])
        )


def kernel(x, dy, gamma):
    del gamma
    m, d = x.shape
    half = m // 2
    n_blk = half // BM

    y_idx = lax.axis_index("y").astype(jnp.int32).reshape((1,))

    grid_spec = pltpu.PrefetchScalarGridSpec(
        num_scalar_prefetch=1,
        grid=(n_blk,),
        in_specs=[
            pl.BlockSpec((BM, d), lambda i, y_ref: (y_ref[0] * n_blk + i, 0)),
            pl.BlockSpec((BM, d), lambda i, y_ref: (y_ref[0] * n_blk + i, 0)),
        ],
        out_specs=pl.BlockSpec((2, d), lambda i, y_ref: (0, 0)),
        scratch_shapes=[
            pltpu.VMEM((2, d), jnp.float32),
            pltpu.VMEM((3, 2, d), jnp.float32),
            pltpu.SemaphoreType.DMA((3,)),
            pltpu.SemaphoreType.DMA((3,)),
        ],
    )

    return pl.pallas_call(
        _body,
        grid_spec=grid_spec,
        out_shape=jax.ShapeDtypeStruct((2, d), jnp.float32),
        compiler_params=pltpu.CompilerParams(collective_id=0),
    )(y_idx, x, dy)


# device time: 18079 ns/iter; 1.1016x vs baseline; 1.1016x over previous
import jax
import jax.numpy as jnp
from jax import lax
from jax.experimental import pallas as pl
from jax.experimental.pallas import tpu as pltpu

BM = 512


def _body(y_ref, x_ref, dy_ref, out_ref, acc_ref, comm_ref, send_sems, recv_sems):
    i = pl.program_id(0)
    n_steps = pl.num_programs(0)

    x = x_ref[...]
    dy = dy_ref[...]
    d = x.shape[1]
    mu = jnp.sum(x, axis=1, keepdims=True) * (1.0 / d)
    xc = x - mu
    var = jnp.sum(xc * xc, axis=1, keepdims=True) * (1.0 / d)
    rstd = lax.rsqrt(var + 1e-5)
    xhat = xc * rstd
    dgamma = jnp.sum(dy * xhat, axis=0, keepdims=True)
    dbeta = jnp.sum(dy, axis=0, keepdims=True)
    part = jnp.concatenate([dgamma, dbeta], axis=0)

    @pl.when(i == 0)
    def _():
        acc_ref[...] = part

    @pl.when(i != 0)
    def _():
        acc_ref[...] += part

    @pl.when(i == n_steps - 1)
    def _():
        my_x = lax.axis_index("x")
        my_y = lax.axis_index("y")
        peers = (
            (1 - my_x, my_y),
            (my_x, 1 - my_y),
            (1 - my_x, 1 - my_y),
        )

        barrier = pltpu.get_barrier_semaphore()
        for peer in peers:
            pl.semaphore_signal(
                barrier, inc=1, device_id=peer,
                device_id_type=pl.DeviceIdType.MESH,
            )
        pl.semaphore_wait(barrier, 3)

        rdmas = []
        for k, peer in enumerate(peers):
            rdma = pltpu.make_async_remote_copy(
                src_ref=acc_ref,
                dst_ref=comm_ref.at[k],
                send_sem=send_sems.at[k],
                recv_sem=recv_sems.at[k],
                device_id=peer,
                device_id_type=pl.DeviceIdType.MESH,
            )
            rdma.start()
            rdmas.append(rdma)
        for rdma in rdmas:
            rdma.wait()

        out_ref[...] = (
            (acc_ref[...] + comm_ref[0]) + (comm_ref[1] + comm_ref[2])
        )


def kernel(x, dy, gamma):
    del gamma
    m, d = x.shape
    half = m // 2
    n_blk = half // BM

    y_idx = lax.axis_index("y").astype(jnp.int32).reshape((1,))

    grid_spec = pltpu.PrefetchScalarGridSpec(
        num_scalar_prefetch=1,
        grid=(n_blk,),
        in_specs=[
            pl.BlockSpec((BM, d), lambda i, y_ref: (y_ref[0] * n_blk + i, 0)),
            pl.BlockSpec((BM, d), lambda i, y_ref: (y_ref[0] * n_blk + i, 0)),
        ],
        out_specs=pl.BlockSpec((2, d), lambda i, y_ref: (0, 0)),
        scratch_shapes=[
            pltpu.VMEM((2, d), jnp.float32),
            pltpu.VMEM((3, 2, d), jnp.float32),
            pltpu.SemaphoreType.DMA((3,)),
            pltpu.SemaphoreType.DMA((3,)),
        ],
    )

    return pl.pallas_call(
        _body,
        grid_spec=grid_spec,
        out_shape=jax.ShapeDtypeStruct((2, d), jnp.float32),
        compiler_params=pltpu.CompilerParams(collective_id=0),
    )(y_idx, x, dy)
